# Initial kernel scaffold; baseline (speedup 1.0000x reference)
import jax
import jax.numpy as jnp
from jax.experimental import pallas as pl


def kernel(x, node_time, edge_index, seed_time, min_timestamp, max_timestamp, update_period, W_self_0, W_neigh_0, b_0, W_self_1, W_neigh_1, b_1, W_head, b_head):
    raise NotImplementedError("write your pallas kernel here")


# speedup vs baseline: 2.1326x; 2.1326x over previous
import jax
import jax.numpy as jnp
from jax import lax
from jax.experimental import pallas as pl
from jax.experimental.pallas import tpu as pltpu
from jax.experimental.pallas import tpu_sc as plsc

N_NODES = 10000
C_DIM = 128
NUM_PERIODS = 4
NC, NS = 2, 16
NB = NUM_PERIODS * NC
K_CHUNK = 128
ALIGN = NS * K_CHUNK
E_EDGES = 320000
E_TOT = E_EDGES + NB * ALIGN
E_TOT_W = E_TOT + 64
EPT = E_EDGES // 32
K_ROUTE = 128
N_RCH = (EPT + K_ROUTE - 1) // K_ROUTE
HALF = 5120
RPC_ACC = 5248
RPT_Z = RPC_ACC // NS
RPT_D = HALF // NS
ROWS_OUT = NC * HALF


def _sc_route():
  mesh = plsc.VectorSubcoreMesh(core_axis_name="c", subcore_axis_name="s",
                                num_cores=NC, num_subcores=NS)
  out_type = [jax.ShapeDtypeStruct((E_TOT_W,), jnp.int32)] * 3
  scratch = [
      pltpu.VMEM((K_ROUTE,), jnp.int32),
      pltpu.VMEM((K_ROUTE,), jnp.int32),
      pltpu.VMEM((K_ROUTE,), jnp.int32),
      pltpu.VMEM((K_ROUTE,), jnp.int32),
      pltpu.VMEM((K_ROUTE,), jnp.int32),
      pltpu.VMEM((K_ROUTE,), jnp.int32),
      pltpu.SemaphoreType.DMA,
      pltpu.SemaphoreType.DMA,
  ]

  def body(e_src, e_dstl, e_pos, e_src2, e_pos2, p_pos, p_vsrc, p_pos2,
           halfbuf, out_src, out_dst, out_src2,
           b_pos, b_src, b_dstl, b_pos2, b_src2, half_v, sem_l, sem_s):
    c = lax.axis_index("c")
    s = lax.axis_index("s")
    wid = c * NS + s
    base = wid * EPT
    pltpu.sync_copy(halfbuf, half_v)

    @pl.loop(0, N_RCH)
    def _route(j):
      off = pl.multiple_of(
          jnp.minimum(base + j * K_ROUTE, base + EPT - K_ROUTE), 8)
      loads = [
          pltpu.async_copy(e_pos.at[pl.ds(off, K_ROUTE)], b_pos, sem_l),
          pltpu.async_copy(e_src.at[pl.ds(off, K_ROUTE)], b_src, sem_l),
          pltpu.async_copy(e_dstl.at[pl.ds(off, K_ROUTE)], b_dstl, sem_l),
          pltpu.async_copy(e_pos2.at[pl.ds(off, K_ROUTE)], b_pos2, sem_l),
          pltpu.async_copy(e_src2.at[pl.ds(off, K_ROUTE)], b_src2, sem_l),
      ]
      for d in loads:
        d.wait()
      stores = [
          pltpu.async_copy(b_src, out_src.at[b_pos], sem_s),
          pltpu.async_copy(b_dstl, out_dst.at[b_pos], sem_s),
          pltpu.async_copy(b_src2, out_src2.at[b_pos2], sem_s),
      ]
      for d in stores:
        d.wait()

    pbase = wid * (NB * ALIGN // 32)
    for j in range(NB * ALIGN // 32 // K_ROUTE):
      off = pbase + j * K_ROUTE
      l1 = pltpu.async_copy(p_pos.at[pl.ds(off, K_ROUTE)], b_pos, sem_l)
      l2 = pltpu.async_copy(p_vsrc.at[pl.ds(off, K_ROUTE)], b_src, sem_l)
      l3 = pltpu.async_copy(p_pos2.at[pl.ds(off, K_ROUTE)], b_pos2, sem_l)
      l1.wait(); l2.wait(); l3.wait()
      s1 = pltpu.async_copy(b_src, out_src.at[b_pos], sem_s)
      s2 = pltpu.async_copy(half_v, out_dst.at[b_pos], sem_s)
      s3 = pltpu.async_copy(half_v, out_src2.at[b_pos2], sem_s)
      s1.wait(); s2.wait(); s3.wait()

  return pl.kernel(
      body, out_type=out_type, mesh=mesh, scratch_types=scratch,
      compiler_params=pltpu.CompilerParams(use_tc_tiling_on_sc=False))


def _sc_segment_sum(with_counts):
  mesh = plsc.VectorSubcoreMesh(core_axis_name="c", subcore_axis_name="s",
                                num_cores=NC, num_subcores=NS)
  out_type = [jax.ShapeDtypeStruct((ROWS_OUT, C_DIM), jnp.float32)]
  scratch = [
      pltpu.VMEM((16,), jnp.int32),
      pltpu.VMEM((K_CHUNK,), jnp.int32),
      pltpu.VMEM((K_CHUNK,), jnp.int32),
      pltpu.VMEM((K_CHUNK, C_DIM), jnp.float32),
      pltpu.VMEM_SHARED((RPC_ACC, C_DIM), jnp.float32),
      pltpu.SemaphoreType.DMA,
  ]
  if with_counts:
    out_type += [jax.ShapeDtypeStruct((ROWS_OUT, 16), jnp.float32)]
    scratch += [
        pltpu.VMEM((K_CHUNK,), jnp.int32),
        pltpu.VMEM((K_CHUNK, 16), jnp.float32),
        pltpu.VMEM((K_CHUNK, 16), jnp.float32),
        pltpu.VMEM_SHARED((RPC_ACC, 16), jnp.float32),
    ]

  def body(*refs):
    if with_counts:
      (tab, srce, dste, srcl2, seg, zrows, zcnt, ones_d_h, ones_s_h,
       out_agg, out_cnt,
       seg_v, src_v, dst_v, rows_v, acc, sem,
       src2_v, ones_d, ones_s, acc_cnt) = refs
    else:
      (tab, srce, dste, seg, zrows,
       out_agg, seg_v, src_v, dst_v, rows_v, acc, sem) = refs
    c = lax.axis_index("c")
    s = lax.axis_index("s")
    zbase = s * RPT_Z
    pltpu.sync_copy(zrows, acc.at[pl.ds(zbase, RPT_Z)])
    if with_counts:
      pltpu.sync_copy(zcnt, acc_cnt.at[pl.ds(zbase, RPT_Z)])
      pltpu.sync_copy(ones_d_h, ones_d)
      pltpu.sync_copy(ones_s_h, ones_s)
    pltpu.sync_copy(seg, seg_v)
    plsc.subcore_barrier()
    vec = seg_v[...]

    def pick(i0, i1):
      return jnp.where(c == 0, vec[i0], vec[i1])

    start = pick(0, 2)
    nch = pick(1, 3)
    base = start + s * nch * K_CHUNK

    def chunk(j, carry):
      off = pl.multiple_of(base + j * K_CHUNK, K_CHUNK)
      pltpu.sync_copy(srce.at[pl.ds(off, K_CHUNK)], src_v)
      pltpu.sync_copy(dste.at[pl.ds(off, K_CHUNK)], dst_v)
      pltpu.async_copy(tab.at[src_v], rows_v, sem).wait()
      pltpu.sync_copy(rows_v, acc.at[dst_v], add=True)
      if with_counts:
        pltpu.sync_copy(ones_d, acc_cnt.at[dst_v], add=True)
      return carry

    lax.fori_loop(0, nch, chunk, 0)
    if with_counts:
      start2 = pick(4, 6)
      nch2 = pick(5, 7)
      base2 = start2 + s * nch2 * K_CHUNK

      def chunk2(j, carry):
        off = pl.multiple_of(base2 + j * K_CHUNK, K_CHUNK)
        pltpu.sync_copy(srcl2.at[pl.ds(off, K_CHUNK)], src2_v)
        pltpu.sync_copy(ones_s, acc_cnt.at[src2_v], add=True)
        return carry

      lax.fori_loop(0, nch2, chunk2, 0)
    plsc.subcore_barrier()
    dbase = s * RPT_D
    obase = c * HALF + dbase
    pltpu.sync_copy(acc.at[pl.ds(dbase, RPT_D)],
                    out_agg.at[pl.ds(obase, RPT_D)])
    if with_counts:
      pltpu.sync_copy(acc_cnt.at[pl.ds(dbase, RPT_D)],
                      out_cnt.at[pl.ds(obase, RPT_D)])

  return pl.kernel(
      body, out_type=out_type, mesh=mesh, scratch_types=scratch,
      compiler_params=pltpu.CompilerParams(use_tc_tiling_on_sc=False))


_BR = 1000


def _tc_layer1(x, agg, cd, ws, wn, b):
  def body(x_r, agg_r, cd_r, ws_r, wn_r, b_r, out_r):
    cnt = cd_r[...][:, 0]
    inv = 1.0 / jnp.maximum(cnt, 1.0)
    agg_n = agg_r[...] * inv[:, None]
    h = (jnp.dot(x_r[...], ws_r[...], preferred_element_type=jnp.float32)
         + jnp.dot(agg_n, wn_r[...], preferred_element_type=jnp.float32)
         + b_r[...])
    out_r[...] = jnp.maximum(h, 0.0)

  grid = (N_NODES // _BR,)
  return pl.pallas_call(
      body,
      grid=grid,
      in_specs=[
          pl.BlockSpec((_BR, C_DIM), lambda i: (i, 0)),
          pl.BlockSpec((_BR, C_DIM), lambda i: (i, 0)),
          pl.BlockSpec((_BR, 16), lambda i: (i, 0)),
          pl.BlockSpec((C_DIM, C_DIM), lambda i: (0, 0)),
          pl.BlockSpec((C_DIM, C_DIM), lambda i: (0, 0)),
          pl.BlockSpec((1, C_DIM), lambda i: (0, 0)),
      ],
      out_specs=pl.BlockSpec((_BR, C_DIM), lambda i: (i, 0)),
      out_shape=jax.ShapeDtypeStruct((N_NODES, C_DIM), jnp.float32),
  )(x, agg, cd, ws, wn, b)


def _tc_layer2(x, h1, agg, cd, ws, wn, b):
  def body(x_r, h1_r, agg_r, cd_r, ws_r, wn_r, b_r, out_r):
    cd = cd_r[...]
    cnt = cd[:, 0]
    inv = 1.0 / jnp.maximum(cnt, 1.0)
    involved = (cnt + cd[:, 1]) > 0.0
    agg_n = agg_r[...] * inv[:, None]
    h = (jnp.dot(h1_r[...], ws_r[...], preferred_element_type=jnp.float32)
         + jnp.dot(agg_n, wn_r[...], preferred_element_type=jnp.float32)
         + b_r[...])
    out_r[...] = x_r[...] + jnp.where(involved[:, None], h, 0.0)

  grid = (N_NODES // _BR,)
  return pl.pallas_call(
      body,
      grid=grid,
      in_specs=[
          pl.BlockSpec((_BR, C_DIM), lambda i: (i, 0)),
          pl.BlockSpec((_BR, C_DIM), lambda i: (i, 0)),
          pl.BlockSpec((_BR, C_DIM), lambda i: (i, 0)),
          pl.BlockSpec((_BR, 16), lambda i: (i, 0)),
          pl.BlockSpec((C_DIM, C_DIM), lambda i: (0, 0)),
          pl.BlockSpec((C_DIM, C_DIM), lambda i: (0, 0)),
          pl.BlockSpec((1, C_DIM), lambda i: (0, 0)),
      ],
      out_specs=pl.BlockSpec((_BR, C_DIM), lambda i: (i, 0)),
      out_shape=jax.ShapeDtypeStruct((N_NODES, C_DIM), jnp.float32),
  )(x, h1, agg, cd, ws, wn, b)


def _tc_head(x, w_row, batch):
  def body(x_r, w_r, out_r):
    out_r[...] = jnp.sum(x_r[...] * w_r[...], axis=1, keepdims=True)

  return pl.pallas_call(
      body,
      grid=(1,),
      in_specs=[
          pl.BlockSpec((batch, C_DIM), lambda i: (0, 0)),
          pl.BlockSpec((1, C_DIM), lambda i: (0, 0)),
      ],
      out_specs=pl.BlockSpec((batch, 1), lambda i: (0, 0)),
      out_shape=jax.ShapeDtypeStruct((batch, 1), jnp.float32),
  )(x, w_row)


def kernel(x, node_time, edge_index, seed_time, min_timestamp, max_timestamp,
           update_period, W_self_0, W_neigh_0, b_0, W_self_1, W_neigh_1, b_1,
           W_head, b_head):
  src = edge_index[0]
  dst = edge_index[1]
  intro = jnp.maximum(node_time[src], node_time[dst])
  per = ((intro - min_timestamp) // update_period).astype(jnp.int32)

  def bucketize(half_id):
    key = per * NC + half_id
    onehots = key[None, :] == jnp.arange(NB, dtype=jnp.int32)[:, None]
    ranks = jnp.cumsum(onehots, axis=1, dtype=jnp.int32)
    cntb = ranks[:, -1]
    lenb = ((cntb + ALIGN - 1) // ALIGN) * ALIGN
    boff = jnp.concatenate(
        [jnp.zeros((1,), jnp.int32), jnp.cumsum(lenb)[:-1].astype(jnp.int32)])
    rank_e = jnp.sum(jnp.where(onehots, ranks - 1, 0), axis=0)
    pos = boff[key] + rank_e
    nchb = (lenb // ALIGN).astype(jnp.int32)
    return boff, cntb, lenb, nchb, pos, onehots

  dhalf = (dst >= HALF).astype(jnp.int32)
  boff, cntb, lenb, nchb, pos, onehots_d = bucketize(dhalf)
  shalf = (src >= HALF).astype(jnp.int32)
  boff2, cntb2, lenb2, nchb2, pos2, _ = bucketize(shalf)

  jj = jnp.arange(ALIGN, dtype=jnp.int32)[None, :]

  def padtab(bo, cn, ln):
    npad = (ln - cn)[:, None]
    ppos = jnp.where(jj < npad, bo[:, None] + cn[:, None] + jj, E_TOT)
    return ppos.reshape(-1).astype(jnp.int32)

  ppos_d = padtab(boff, cntb, lenb)
  ppos_s = padtab(boff2, cntb2, lenb2)
  first_src = src[jnp.argmax(onehots_d, axis=1)]
  p_vsrc = jnp.broadcast_to(first_src[:, None], (NB, ALIGN)).reshape(-1)
  halfbuf = jnp.full((K_ROUTE,), HALF, jnp.int32)

  route = _sc_route()
  src_p, dst_p, src_p2 = route(src, dst - dhalf * HALF, pos,
                               src - shalf * HALF, pos2,
                               ppos_d, p_vsrc, ppos_s, halfbuf)

  seg_all = jnp.zeros((NUM_PERIODS, 16), jnp.int32)
  bo = boff.reshape(NUM_PERIODS, NC)
  nc_ = nchb.reshape(NUM_PERIODS, NC)
  bo2 = boff2.reshape(NUM_PERIODS, NC)
  nc2_ = nchb2.reshape(NUM_PERIODS, NC)
  seg_all = (seg_all.at[:, 0].set(bo[:, 0]).at[:, 1].set(nc_[:, 0])
             .at[:, 2].set(bo[:, 1]).at[:, 3].set(nc_[:, 1])
             .at[:, 4].set(bo2[:, 0]).at[:, 5].set(nc2_[:, 0])
             .at[:, 6].set(bo2[:, 1]).at[:, 7].set(nc2_[:, 1]))

  zrows = jnp.zeros((RPT_Z, C_DIM), jnp.float32)
  zcnt = jnp.zeros((RPT_Z, 16), jnp.float32)
  ones_d_pat = jnp.zeros((K_CHUNK, 16), jnp.float32).at[:, 0].set(1.0)
  ones_s_pat = jnp.zeros((K_CHUNK, 16), jnp.float32).at[:, 1].set(1.0)

  sc0 = _sc_segment_sum(with_counts=True)
  sc1 = _sc_segment_sum(with_counts=False)

  b0r = b_0.reshape(1, C_DIM)
  b1r = b_1.reshape(1, C_DIM)

  xc = x
  for p in range(NUM_PERIODS):
    segp = seg_all[p]
    agg0, cd = sc0(xc, src_p, dst_p, src_p2, segp, zrows, zcnt, ones_d_pat,
                   ones_s_pat)
    h1 = _tc_layer1(xc, agg0, cd, W_self_0, W_neigh_0, b0r)
    (agg1,) = sc1(h1, src_p, dst_p, segp, zrows)
    xc = _tc_layer2(xc, h1, agg1, cd, W_self_1, W_neigh_1, b1r)

  batch = int(seed_time.shape[0])
  w_row = W_head.reshape(1, C_DIM)
  out = _tc_head(xc, w_row, batch) + b_head
  return out
